# baseline (speedup 1.0000x reference)
import functools

import jax
import jax.numpy as jnp
from jax import lax
from jax.experimental import pallas as pl
from jax.experimental.pallas import tpu as pltpu
from jax.experimental.pallas import tpu_sc as plsc

_NC = 2
_NS = 16
_NW = _NC * _NS
_L = 16
_CHUNK = 128
_NBLK = 8192
_QCH = _NBLK // 4


def _pack_body(x_ref, o_ref):
    eye = jnp.eye(32, dtype=jnp.float32)
    for k in range(4):
        xc = x_ref[:, k * _QCH:(k + 1) * _QCH]
        o_ref[:, k * 32:(k + 1) * 32] = lax.dot_general(
            xc, eye, (((0,), (0,)), ((), ())),
            preferred_element_type=jnp.float32)


def _pack_table(table):
    D, V = table.shape
    nblocks = -(-V // _NBLK)
    return pl.pallas_call(
        _pack_body,
        grid=(nblocks,),
        in_specs=[pl.BlockSpec((D, _NBLK), lambda i: (0, i))],
        out_specs=pl.BlockSpec((_QCH, 4 * D), lambda i: (i, 0)),
        out_shape=jax.ShapeDtypeStruct((nblocks * _QCH, 4 * D), jnp.float32),
        compiler_params=pltpu.CompilerParams(
            fuse_transposed_lhs_in_matmul=True),
    )(table)


def _sc_scores(user_ids, item_ids, neg_item_ids, utab, itab):
    B = user_ids.shape[0]
    D = 32
    bpw = B // _NW
    nchunk = bpw // _CHUNK

    uids3 = user_ids.reshape(_NW, nchunk, _CHUNK)
    pids3 = item_ids.reshape(_NW, nchunk, _CHUNK)
    nids3 = neg_item_ids.reshape(_NW, nchunk, _CHUNK)

    mesh = plsc.VectorSubcoreMesh(core_axis_name="c", subcore_axis_name="s")

    @functools.partial(
        pl.kernel,
        out_type=jax.ShapeDtypeStruct((B,), jnp.float32),
        mesh=mesh,
        compiler_params=pltpu.CompilerParams(needs_layout_passes=False),
        scratch_types=[
            pltpu.VMEM((nchunk, _CHUNK), jnp.int32),
            pltpu.VMEM((nchunk, _CHUNK), jnp.int32),
            pltpu.VMEM((nchunk, _CHUNK), jnp.int32),
            pltpu.VMEM((nchunk, _CHUNK), jnp.int32),
            pltpu.VMEM((nchunk, _CHUNK), jnp.int32),
            pltpu.VMEM((nchunk, _CHUNK), jnp.int32),
            pltpu.VMEM((2, _CHUNK, 4 * D), jnp.float32),
            pltpu.VMEM((2, _CHUNK, 4 * D), jnp.float32),
            pltpu.VMEM((2, _CHUNK, 4 * D), jnp.float32),
            pltpu.VMEM((bpw,), jnp.float32),
            pltpu.SemaphoreType.DMA,
            pltpu.SemaphoreType.DMA,
        ],
    )
    def sc_kernel(uids_hbm, pids_hbm, nids_hbm, utab_hbm, itab_hbm, out_hbm,
                  uidx_v, pidx_v, nidx_v, ublk_v, pblk_v, nblk_v,
                  ubuf, pbuf, nbuf, t_v, sem0, sem1):
        wid = lax.axis_index("s") * _NC + lax.axis_index("c")
        sems = (sem0, sem1)

        pltpu.sync_copy(uids_hbm.at[wid], uidx_v)
        pltpu.sync_copy(pids_hbm.at[wid], pidx_v)
        pltpu.sync_copy(nids_hbm.at[wid], nidx_v)

        def rowix(ids):
            return (lax.shift_left(lax.shift_right_logical(ids, 13), 11)
                    + (ids & (_QCH - 1)))

        def blk_body(j, carry):
            c = j // (_CHUNK // _L)
            g = j % (_CHUNK // _L)
            sl = pl.ds(g * _L, _L)
            ublk_v[c, sl] = rowix(uidx_v[c, sl])
            pblk_v[c, sl] = rowix(pidx_v[c, sl])
            nblk_v[c, sl] = rowix(nidx_v[c, sl])
            return carry

        lax.fori_loop(0, nchunk * (_CHUNK // _L), blk_body, 0)

        def fire(c, slot):
            return [
                pltpu.async_copy(utab_hbm.at[ublk_v.at[c]], ubuf.at[slot],
                                 sems[slot]),
                pltpu.async_copy(itab_hbm.at[pblk_v.at[c]], pbuf.at[slot],
                                 sems[slot]),
                pltpu.async_copy(itab_hbm.at[nblk_v.at[c]], nbuf.at[slot],
                                 sems[slot]),
            ]

        iota = lax.iota(jnp.int32, _L)
        pending = {0: fire(0, 0)}
        for c in range(nchunk):
            if c + 1 < nchunk:
                pending[c + 1] = fire(c + 1, (c + 1) % 2)
            for cp in pending.pop(c):
                cp.wait()
            slot = c % 2
            ub, pb, nb = ubuf.at[slot], pbuf.at[slot], nbuf.at[slot]

            def body(g, carry, c=c, ub=ub, pb=pb, nb=nb):
                sl = pl.ds(g * _L, _L)
                rows = g * _L + iota
                cu = (lax.shift_right_logical(uidx_v[c, sl], 11) & 3) << 5
                cp_ = (lax.shift_right_logical(pidx_v[c, sl], 11) & 3) << 5
                cn = (lax.shift_right_logical(nidx_v[c, sl], 11) & 3) << 5
                acc = jnp.zeros((_L,), jnp.float32)
                for d in range(D):
                    uu = plsc.load_gather(ub, [rows, cu + d])
                    pp = plsc.load_gather(pb, [rows, cp_ + d])
                    nn = plsc.load_gather(nb, [rows, cn + d])
                    acc = acc + uu * (pp - nn)
                t_v[pl.ds(c * _CHUNK + g * _L, _L)] = acc
                return carry

            lax.fori_loop(0, _CHUNK // _L, body, 0)

        pltpu.sync_copy(t_v, out_hbm.at[pl.ds(wid * bpw, bpw)])

    return sc_kernel(uids3, pids3, nids3, utab, itab)


def _tc_loss_body(x_ref, o_ref):
    x = x_ref[...]
    ls = jnp.minimum(x, 0.0) - jnp.log1p(jnp.exp(-jnp.abs(x)))
    o_ref[...] = jnp.broadcast_to(-jnp.mean(ls), (1, 1))


def kernel(user_ids, item_ids, neg_item_ids, user_table, item_table):
    utab = _pack_table(user_table.T)
    itab = _pack_table(item_table.T)
    scores = _sc_scores(user_ids, item_ids, neg_item_ids, utab, itab)
    B = scores.shape[0]
    loss2d = pl.pallas_call(
        _tc_loss_body,
        out_shape=jax.ShapeDtypeStruct((1, 1), jnp.float32),
    )(scores.reshape(128, B // 128))
    return loss2d[0, 0]


# speedup vs baseline: 1.4554x; 1.4554x over previous
import functools

import jax
import jax.numpy as jnp
from jax import lax
from jax.experimental import pallas as pl
from jax.experimental.pallas import tpu as pltpu
from jax.experimental.pallas import tpu_sc as plsc

_NC = 2
_NS = 16
_NW = _NC * _NS
_L = 16
_TB = 8


def _sc_scores(user_ids, item_ids, neg_item_ids, user_table, item_table):
    B = user_ids.shape[0]
    V, D = user_table.shape
    bpw = B // _NW
    ngroup = bpw // _L

    uids2 = user_ids.reshape(_NW, bpw)
    pids2 = item_ids.reshape(_NW, bpw)
    nids2 = neg_item_ids.reshape(_NW, bpw)
    utab3 = user_table.reshape(V // _TB, _TB, D)
    itab3 = item_table.reshape(V // _TB, _TB, D)

    mesh = plsc.VectorSubcoreMesh(core_axis_name="c", subcore_axis_name="s")

    @functools.partial(
        pl.kernel,
        out_type=jax.ShapeDtypeStruct((B,), jnp.float32),
        mesh=mesh,
        compiler_params=pltpu.CompilerParams(needs_layout_passes=False),
        scratch_types=[
            pltpu.VMEM((bpw,), jnp.int32),
            pltpu.VMEM((bpw,), jnp.int32),
            pltpu.VMEM((bpw,), jnp.int32),
            pltpu.VMEM((bpw // 2, D), jnp.float32),
            pltpu.VMEM((bpw // 2, D), jnp.float32),
            pltpu.VMEM((bpw // 2, D), jnp.float32),
            pltpu.VMEM((bpw,), jnp.float32),
            pltpu.SemaphoreType.DMA,
            pltpu.SemaphoreType.DMA,
            pltpu.SemaphoreType.DMA,
        ],
    )
    def sc_kernel(uids_hbm, pids_hbm, nids_hbm, utab_hbm, itab_hbm, out_hbm,
                  uidx_v, pidx_v, nidx_v, u_v, p_v, n_v, t_v,
                  sem_u, sem_p, sem_n):
        wid = lax.axis_index("s") * _NC + lax.axis_index("c")

        pltpu.sync_copy(uids_hbm.at[wid], uidx_v)
        pltpu.sync_copy(pids_hbm.at[wid], pidx_v)
        pltpu.sync_copy(nids_hbm.at[wid], nidx_v)

        iota = lax.iota(jnp.int32, _L)

        for h in range(2):
            hbase = h * (ngroup // 2)

            def fire_body(g, carry, hbase=hbase):
                sl = pl.ds((hbase + g) * _L, _L)
                uu = uidx_v[sl]
                pp = pidx_v[sl]
                nn = nidx_v[sl]
                for j in range(_L):
                    r = g * _L + j
                    pltpu.async_copy(
                        utab_hbm.at[uu[j] >> 3, uu[j] & (_TB - 1)],
                        u_v.at[r], sem_u)
                    pltpu.async_copy(
                        itab_hbm.at[pp[j] >> 3, pp[j] & (_TB - 1)],
                        p_v.at[r], sem_p)
                    pltpu.async_copy(
                        itab_hbm.at[nn[j] >> 3, nn[j] & (_TB - 1)],
                        n_v.at[r], sem_n)
                return carry

            lax.fori_loop(0, ngroup // 2, fire_body, 0)

            def drain_body(g, carry):
                for j in range(_L):
                    r = g * _L + j
                    pltpu.make_async_copy(utab_hbm.at[0, 0],
                                          u_v.at[r], sem_u).wait()
                    pltpu.make_async_copy(itab_hbm.at[0, 0],
                                          p_v.at[r], sem_p).wait()
                    pltpu.make_async_copy(itab_hbm.at[0, 0],
                                          n_v.at[r], sem_n).wait()
                return carry

            lax.fori_loop(0, ngroup // 2, drain_body, 0)

            def body(g, carry, hbase=hbase):
                rows = g * _L + iota
                acc = jnp.zeros((_L,), jnp.float32)
                for d in range(D):
                    dcol = jnp.full((_L,), d, jnp.int32)
                    uu = plsc.load_gather(u_v, [rows, dcol])
                    pp = plsc.load_gather(p_v, [rows, dcol])
                    nn = plsc.load_gather(n_v, [rows, dcol])
                    acc = acc + uu * (pp - nn)
                t_v[pl.ds((hbase + g) * _L, _L)] = acc
                return carry

            lax.fori_loop(0, ngroup // 2, body, 0)

        pltpu.sync_copy(t_v, out_hbm.at[pl.ds(wid * bpw, bpw)])

    return sc_kernel(uids2, pids2, nids2, utab3, itab3)


def _tc_loss_body(x_ref, o_ref):
    x = x_ref[...]
    ls = jnp.minimum(x, 0.0) - jnp.log1p(jnp.exp(-jnp.abs(x)))
    o_ref[...] = jnp.broadcast_to(-jnp.mean(ls), (1, 1))


def kernel(user_ids, item_ids, neg_item_ids, user_table, item_table):
    scores = _sc_scores(user_ids, item_ids, neg_item_ids,
                        user_table, item_table)
    B = scores.shape[0]
    loss2d = pl.pallas_call(
        _tc_loss_body,
        out_shape=jax.ShapeDtypeStruct((1, 1), jnp.float32),
    )(scores.reshape(128, B // 128))
    return loss2d[0, 0]
